# baseline (speedup 1.0000x reference)
import functools

import jax
import jax.numpy as jnp
from jax import lax
from jax.experimental import pallas as pl
from jax.experimental.pallas import tpu as pltpu
from jax.experimental.pallas import tpu_sc as plsc

N_W_ROWS = 4096
N_W_COLS = 4096
NWORKERS = 32
ROWS_PER_W = N_W_ROWS // NWORKERS
RB = 8
NB = ROWS_PER_W // RB


def _sc_scan_body(w_hbm, out_hbm, wbuf, cnt_buf, sems):
    wid = lax.axis_index("s") * 2 + lax.axis_index("c")
    base = wid * ROWS_PER_W

    def w_copy(b, slot):
        return pltpu.make_async_copy(
            w_hbm.at[pl.ds(base + b * RB, RB), :], wbuf.at[slot], sems.at[slot]
        )

    w_copy(0, 0).start()
    cnt = jnp.zeros((16,), jnp.float32)
    for b in range(NB):
        slot = b % 2
        w_copy(b, slot).wait()
        if b + 1 < NB:
            w_copy(b + 1, 1 - slot).start()

        def chunk(k, acc, _slot=slot):
            r = k // 256
            c = k % 256
            v = wbuf[_slot, r, pl.ds(c * 16, 16)]
            return acc + jnp.where(v != 0.0, 1.0, 0.0)

        cnt = lax.fori_loop(0, RB * 256, chunk, cnt, unroll=8)

    cnt_buf[...] = cnt
    pltpu.sync_copy(cnt_buf, out_hbm.at[wid])


def _sc_scan(W):
    mesh = plsc.VectorSubcoreMesh(core_axis_name="c", subcore_axis_name="s")
    return pl.kernel(
        _sc_scan_body,
        out_type=jax.ShapeDtypeStruct((NWORKERS, 16), jnp.float32),
        mesh=mesh,
        scratch_types=[
            pltpu.VMEM((2, RB, N_W_COLS), jnp.float32),
            pltpu.VMEM((16,), jnp.float32),
            pltpu.SemaphoreType.DMA((2,)),
        ],
    )(W)


@jax.jit
def kernel(X, W):
    cnts = _sc_scan(W)
    return jnp.zeros((X.shape[0], W.shape[0]), jnp.float32) + cnts.sum()


# speedup vs baseline: 2.5509x; 2.5509x over previous
import jax
import jax.numpy as jnp
from jax.experimental import pallas as pl
from jax.experimental.pallas import tpu as pltpu

TN = 512


def _matmul_kernel(x_ref, w_ref, o_ref):
    o_ref[...] = jax.lax.dot_general(
        x_ref[...], w_ref[...],
        dimension_numbers=(((1,), (1,)), ((), ())),
        preferred_element_type=jnp.float32,
    )


@jax.jit
def kernel(X, W):
    batch, n_in = X.shape
    n_out = W.shape[0]
    grid = (n_out // TN,)
    return pl.pallas_call(
        _matmul_kernel,
        grid=grid,
        in_specs=[
            pl.BlockSpec((batch, n_in), lambda j: (0, 0)),
            pl.BlockSpec((TN, n_in), lambda j: (j, 0)),
        ],
        out_specs=pl.BlockSpec((batch, TN), lambda j: (0, j)),
        out_shape=jax.ShapeDtypeStruct((batch, n_out), jnp.float32),
        compiler_params=pltpu.CompilerParams(
            dimension_semantics=("arbitrary",),
        ),
    )(X, W)


# speedup vs baseline: 2.5525x; 1.0006x over previous
import jax
import jax.numpy as jnp
from jax.experimental import pallas as pl
from jax.experimental.pallas import tpu as pltpu

TN = 512


def _matmul_kernel(x_ref, w_ref, o_ref):
    o_ref[...] = jax.lax.dot_general(
        x_ref[...], w_ref[...],
        dimension_numbers=(((1,), (1,)), ((), ())),
        preferred_element_type=jnp.float32,
    )


@jax.jit
def kernel(X, W):
    batch, n_in = X.shape
    n_out = W.shape[0]
    grid = (n_out // TN,)
    return pl.pallas_call(
        _matmul_kernel,
        grid=grid,
        in_specs=[
            pl.BlockSpec((batch, n_in), lambda j: (0, 0)),
            pl.BlockSpec((TN, n_in), lambda j: (j, 0)),
        ],
        out_specs=pl.BlockSpec((batch, TN), lambda j: (0, j)),
        out_shape=jax.ShapeDtypeStruct((batch, n_out), jnp.float32),
        compiler_params=pltpu.CompilerParams(
            dimension_semantics=("parallel",),
        ),
    )(X, W)
